# baseline (speedup 1.0000x reference)
import functools

import jax
import jax.numpy as jnp
from jax import lax
from jax.experimental import pallas as pl
from jax.experimental.pallas import tpu as pltpu
from jax.experimental.pallas import tpu_sc as plsc

BATCH = 128
SEQ = 4096
DIM = 128
LANES = 16
NUM_CORES = 2
NUM_SUBCORES = 16
NW = NUM_CORES * NUM_SUBCORES
CHUNK = SEQ // NW
CHUNK_BYTES = CHUNK * DIM * 4


def _make_sc_call():
    mesh = plsc.VectorSubcoreMesh(core_axis_name="c", subcore_axis_name="s")

    @functools.partial(
        pl.kernel,
        mesh=mesh,
        out_type=jax.ShapeDtypeStruct((BATCH * SEQ * DIM,), jnp.float32),
        scratch_types=[
            pltpu.VMEM((BATCH + LANES,), jnp.int32),
            pltpu.VMEM((CHUNK * DIM,), jnp.float32),
            pltpu.VMEM((CHUNK * DIM,), jnp.float32),
            pltpu.SemaphoreType.DMA,
        ],
    )
    def sc_positional(len_hbm, table_hbm, out_hbm, len_v, tab_v, fill_v, sem):
        cid = lax.axis_index("c")
        sid = lax.axis_index("s")
        wid = sid * NUM_CORES + cid
        t0 = wid * CHUNK

        pltpu.sync_copy(len_hbm, len_v.at[pl.ds(0, BATCH)])
        pltpu.sync_copy(table_hbm.at[pl.ds(t0 * DIM, CHUNK * DIM)], tab_v)
        pltpu.sync_copy(table_hbm.at[pl.ds(0, DIM)], fill_v.at[pl.ds(0, DIM)])

        def full_body(b, carry):
            blen = len_v[pl.ds(b, LANES)][0]

            @pl.when(blen - t0 >= CHUNK)
            def _full():
                pltpu.async_copy(
                    tab_v,
                    out_hbm.at[pl.ds((b * SEQ + t0) * DIM, CHUNK * DIM)], sem)

            return carry

        lax.fori_loop(0, BATCH, full_body, 0)

        row0 = [fill_v[pl.ds(g * LANES, LANES)] for g in range(DIM // LANES)]

        def fill_body(r, carry):
            for g in range(DIM // LANES):
                fill_v[pl.ds(r * DIM + g * LANES, LANES)] = row0[g]
            return carry

        lax.fori_loop(1, CHUNK, fill_body, 0)

        def batch_body(b, carry):
            blen = len_v[pl.ds(b, LANES)][0]
            split = jnp.clip(blen - t0, 0, CHUNK)
            base = b * SEQ + t0

            @pl.when(split == 0)
            def _fill():
                pltpu.async_copy(
                    fill_v, out_hbm.at[pl.ds(base * DIM, CHUNK * DIM)], sem)

            @pl.when((split > 0) & (split < CHUNK))
            def _boundary():
                pos = jnp.int32(0)
                for k in (64, 32, 16, 8, 4, 2, 1):
                    part = split & k

                    @pl.when(part != 0)
                    def _pre(pos=pos, k=k):
                        pltpu.async_copy(
                            tab_v.at[pl.ds(pos * DIM, k * DIM)],
                            out_hbm.at[pl.ds((base + pos) * DIM, k * DIM)],
                            sem)

                    pos = pos + part
                rem = CHUNK - split
                for k in (64, 32, 16, 8, 4, 2, 1):
                    part = rem & k

                    @pl.when(part != 0)
                    def _suf(pos=pos, k=k):
                        pltpu.async_copy(
                            fill_v.at[pl.ds(0, k * DIM)],
                            out_hbm.at[pl.ds((base + pos) * DIM, k * DIM)],
                            sem)

                    pos = pos + part

            return carry

        lax.fori_loop(0, BATCH, batch_body, 0)

        def drain_body(b, carry):
            pltpu.make_async_copy(
                out_hbm.at[pl.ds(0, CHUNK * DIM)], tab_v, sem).wait()
            return carry

        lax.fori_loop(0, BATCH, drain_body, 0)

    return sc_positional


_sc_call = _make_sc_call()


def kernel(batch_lengths, max_length, encoding_weight):
    lengths = jnp.minimum(batch_lengths.astype(jnp.int32), max_length)
    flat = _sc_call(lengths, encoding_weight.reshape(SEQ * DIM))
    return flat.reshape(BATCH, SEQ, DIM)


# speedup vs baseline: 1.0248x; 1.0248x over previous
import functools

import jax
import jax.numpy as jnp
from jax import lax
from jax.experimental import pallas as pl
from jax.experimental.pallas import tpu as pltpu
from jax.experimental.pallas import tpu_sc as plsc

BATCH = 128
SEQ = 4096
DIM = 128
LANES = 16
NUM_CORES = 2
NUM_SUBCORES = 16
NW = NUM_CORES * NUM_SUBCORES
CHUNK = SEQ // NW
CHUNK_BYTES = CHUNK * DIM * 4


def _make_sc_call():
    mesh = plsc.VectorSubcoreMesh(core_axis_name="c", subcore_axis_name="s")

    @functools.partial(
        pl.kernel,
        mesh=mesh,
        out_type=jax.ShapeDtypeStruct((BATCH * SEQ * DIM,), jnp.float32),
        scratch_types=[
            pltpu.VMEM((BATCH + LANES,), jnp.int32),
            pltpu.VMEM((CHUNK * DIM,), jnp.float32),
            pltpu.VMEM((CHUNK * DIM,), jnp.float32),
            pltpu.SemaphoreType.DMA,
        ],
    )
    def sc_positional(len_hbm, table_hbm, out_hbm, len_v, tab_v, fill_v, sem):
        cid = lax.axis_index("c")
        sid = lax.axis_index("s")
        wid = sid * NUM_CORES + cid
        t0 = wid * CHUNK

        pltpu.sync_copy(len_hbm, len_v.at[pl.ds(0, BATCH)])
        pltpu.sync_copy(table_hbm.at[pl.ds(t0 * DIM, CHUNK * DIM)], tab_v)
        pltpu.sync_copy(table_hbm.at[pl.ds(0, DIM)], fill_v.at[pl.ds(0, DIM)])
        row0 = [fill_v[pl.ds(g * LANES, LANES)] for g in range(DIM // LANES)]

        def fill_body(r, carry):
            for g in range(DIM // LANES):
                fill_v[pl.ds(r * DIM + g * LANES, LANES)] = row0[g]
            return carry

        lax.fori_loop(1, CHUNK, fill_body, 0)

        def batch_body(b, carry):
            blen = len_v[pl.ds(b, LANES)][0]
            split = jnp.clip(blen - t0, 0, CHUNK)
            base = b * SEQ + t0

            @pl.when(split == CHUNK)
            def _full():
                pltpu.async_copy(
                    tab_v, out_hbm.at[pl.ds(base * DIM, CHUNK * DIM)], sem)

            @pl.when(split == 0)
            def _fill():
                pltpu.async_copy(
                    fill_v, out_hbm.at[pl.ds(base * DIM, CHUNK * DIM)], sem)

            @pl.when((split > 0) & (split < CHUNK))
            def _boundary():
                pos = jnp.int32(0)
                for k in (64, 32, 16, 8, 4, 2, 1):
                    part = split & k

                    @pl.when(part != 0)
                    def _pre(pos=pos, k=k):
                        pltpu.async_copy(
                            tab_v.at[pl.ds(pos * DIM, k * DIM)],
                            out_hbm.at[pl.ds((base + pos) * DIM, k * DIM)],
                            sem)

                    pos = pos + part
                rem = CHUNK - split
                for k in (64, 32, 16, 8, 4, 2, 1):
                    part = rem & k

                    @pl.when(part != 0)
                    def _suf(pos=pos, k=k):
                        pltpu.async_copy(
                            fill_v.at[pl.ds(0, k * DIM)],
                            out_hbm.at[pl.ds((base + pos) * DIM, k * DIM)],
                            sem)

                    pos = pos + part

            return carry

        lax.fori_loop(0, BATCH, batch_body, 0)

        def drain_body(b, carry):
            pltpu.make_async_copy(
                out_hbm.at[pl.ds(0, CHUNK * DIM)], tab_v, sem).wait()
            return carry

        lax.fori_loop(0, BATCH, drain_body, 0)

    return sc_positional


_sc_call = _make_sc_call()


def kernel(batch_lengths, max_length, encoding_weight):
    lengths = jnp.minimum(batch_lengths.astype(jnp.int32), max_length)
    flat = _sc_call(lengths, encoding_weight.reshape(SEQ * DIM))
    return flat.reshape(BATCH, SEQ, DIM)


# speedup vs baseline: 1.0273x; 1.0025x over previous
import functools

import jax
import jax.numpy as jnp
from jax import lax
from jax.experimental import pallas as pl
from jax.experimental.pallas import tpu as pltpu
from jax.experimental.pallas import tpu_sc as plsc

BATCH = 128
SEQ = 4096
DIM = 128
LANES = 16
NUM_CORES = 2
NUM_SUBCORES = 16
NW = NUM_CORES * NUM_SUBCORES
CHUNK = SEQ // NW


def _make_sc_call():
    mesh = plsc.VectorSubcoreMesh(core_axis_name="c", subcore_axis_name="s")

    @functools.partial(
        pl.kernel,
        mesh=mesh,
        out_type=jax.ShapeDtypeStruct((BATCH * SEQ * DIM,), jnp.float32),
        scratch_types=[
            pltpu.VMEM((BATCH + LANES,), jnp.int32),
            pltpu.VMEM((CHUNK * DIM,), jnp.float32),
            pltpu.VMEM((CHUNK * DIM,), jnp.float32),
            pltpu.SemaphoreType.DMA,
        ],
    )
    def sc_positional(len_hbm, table_hbm, out_hbm, len_v, tab_v, fill_v, sem):
        cid = lax.axis_index("c")
        sid = lax.axis_index("s")
        wid = sid * NUM_CORES + cid
        t0 = wid * CHUNK

        pltpu.sync_copy(len_hbm, len_v.at[pl.ds(0, BATCH)])
        pltpu.sync_copy(table_hbm.at[pl.ds(t0 * DIM, CHUNK * DIM)], tab_v)
        pltpu.sync_copy(table_hbm.at[pl.ds(0, DIM)], fill_v.at[pl.ds(0, DIM)])
        row0 = [fill_v[pl.ds(g * LANES, LANES)] for g in range(DIM // LANES)]

        def fill_body(r, carry):
            for g in range(DIM // LANES):
                fill_v[pl.ds(r * DIM + g * LANES, LANES)] = row0[g]
            return carry

        lax.fori_loop(1, CHUNK, fill_body, 0)

        def batch_body(b, carry):
            blen = len_v[pl.ds(b, LANES)][0]
            split = jnp.clip(blen - t0, 0, CHUNK)
            base = b * SEQ + t0

            @pl.when(split == CHUNK)
            def _full():
                pltpu.async_copy(
                    tab_v, out_hbm.at[pl.ds(base * DIM, CHUNK * DIM)], sem)

            @pl.when(split == 0)
            def _fill():
                pltpu.async_copy(
                    fill_v, out_hbm.at[pl.ds(base * DIM, CHUNK * DIM)], sem)

            @pl.when((split > 0) & (split < CHUNK))
            def _boundary():
                pos = jnp.int32(0)
                for k in (64, 32, 16, 8, 4, 2, 1):
                    part = split & k

                    @pl.when(part != 0)
                    def _pre(pos=pos, k=k):
                        pltpu.async_copy(
                            tab_v.at[pl.ds(pos * DIM, k * DIM)],
                            out_hbm.at[pl.ds((base + pos) * DIM, k * DIM)],
                            sem)

                    pos = pos + part
                rem = CHUNK - split
                for k in (64, 32, 16, 8, 4, 2, 1):
                    part = rem & k

                    @pl.when(part != 0)
                    def _suf(pos=pos, k=k):
                        pltpu.async_copy(
                            fill_v.at[pl.ds(0, k * DIM)],
                            out_hbm.at[pl.ds((base + pos) * DIM, k * DIM)],
                            sem)

                    pos = pos + part

            return carry

        lax.fori_loop(0, BATCH, batch_body, 0)

        def drain_body(b, carry):
            pltpu.make_async_copy(
                out_hbm.at[pl.ds(0, CHUNK * DIM)], tab_v, sem).wait()
            return carry

        lax.fori_loop(0, BATCH, drain_body, 0)

    return sc_positional


_sc_call = _make_sc_call()


def kernel(batch_lengths, max_length, encoding_weight):
    lengths = jnp.minimum(batch_lengths.astype(jnp.int32), max_length)
    flat = _sc_call(lengths, encoding_weight.reshape(SEQ * DIM))
    return flat.reshape(BATCH, SEQ, DIM)
